# baseline (speedup 1.0000x reference)
import functools

import jax
import jax.numpy as jnp
from jax import lax
from jax.experimental import pallas as pl
from jax.experimental.pallas import tpu as pltpu
from jax.experimental.pallas import tpu_sc as plsc

_N = 4096
_V = 32000
_LS = 0.1
_RB = 1024
_CB = 6400

_NW = 32
_TPW = _N // _NW
_RPR = _V // 128


@functools.partial(
    pl.kernel,
    mesh=plsc.VectorSubcoreMesh(core_axis_name="c", subcore_axis_name="s"),
    out_type=jax.ShapeDtypeStruct((_NW, 16), jnp.float32),
    scratch_types=[
        pltpu.VMEM((_TPW,), jnp.int32),
        pltpu.VMEM((_TPW,), jnp.int32),
        pltpu.VMEM((_TPW,), jnp.float32),
        pltpu.VMEM((16,), jnp.float32),
        pltpu.SemaphoreType.DMA,
    ],
)
def _sc_gather(lp_hbm, tgt_hbm, out_hbm, tgt_v, idx_v, vals_v, acc_v, sem):
    wid = lax.axis_index("s") * 2 + lax.axis_index("c")
    base = wid * _TPW
    pltpu.sync_copy(tgt_hbm.at[pl.ds(base, _TPW)], tgt_v)
    for g in range(_TPW // 16):
        t = tgt_v[pl.ds(g * 16, 16)]
        tc = jnp.maximum(t, 0)
        ig = base + g * 16 + lax.iota(jnp.int32, 16)
        idx_v[pl.ds(g * 16, 16)] = ig * _V + tc
    pltpu.async_copy(lp_hbm.at[idx_v], vals_v, sem).wait()
    acc = jnp.zeros((16,), jnp.float32)
    for g in range(_TPW // 16):
        t = tgt_v[pl.ds(g * 16, 16)]
        vals = vals_v[pl.ds(g * 16, 16)]
        acc = acc + jnp.where(t != 1, vals, 0.0)
    acc_v[...] = acc
    pltpu.sync_copy(acc_v, out_hbm.at[wid])


def _tc_body(tgt_ref, lp_ref, out_ref, acc_ref):
    i = pl.program_id(0)
    j = pl.program_id(1)

    @pl.when((i == 0) & (j == 0))
    def _init():
        acc_ref[0] = 0.0
        acc_ref[1] = 0.0

    valid = (tgt_ref[...] != 1).astype(jnp.float32)
    rowsum = jnp.sum(lp_ref[...], axis=1, keepdims=True)
    acc_ref[0] += jnp.sum(rowsum * valid)

    @pl.when(j == 0)
    def _count():
        acc_ref[1] += jnp.sum(valid)

    @pl.when((i == pl.num_programs(0) - 1) & (j == pl.num_programs(1) - 1))
    def _finalize():
        out_ref[0, 0] = acc_ref[0]
        out_ref[0, 1] = acc_ref[1]


def kernel(log_probs, targets, triplets):
    tgt = jnp.asarray(targets, jnp.int32)
    s2_partials = _sc_gather(log_probs.reshape(_N * _V), tgt)
    out = pl.pallas_call(
        _tc_body,
        grid=(_N // _RB, _V // _CB),
        in_specs=[
            pl.BlockSpec((_RB, 1), lambda i, j: (i, 0)),
            pl.BlockSpec((_RB, _CB), lambda i, j: (i, j)),
        ],
        out_specs=pl.BlockSpec(memory_space=pltpu.SMEM),
        out_shape=jax.ShapeDtypeStruct((1, 2), jnp.float32),
        scratch_shapes=[pltpu.SMEM((2,), jnp.float32)],
    )(tgt.reshape(_N, 1), log_probs)
    c1 = _LS / _V
    c2 = 1.0 - _LS - c1
    s2 = jnp.sum(s2_partials)
    return -(c1 * out[0, 0] + c2 * s2) / jnp.maximum(out[0, 1], 1.0)


# speedup vs baseline: 3.1223x; 3.1223x over previous
import jax
import jax.numpy as jnp
from jax.experimental import pallas as pl
from jax.experimental.pallas import tpu as pltpu

_N = 4096
_V = 32000
_LS = 0.1
_RB = 1024
_CB = 6400


def _body(tgt_ref, col_ref, lp_ref, out_ref, acc_ref):
    i = pl.program_id(0)
    j = pl.program_id(1)

    @pl.when((i == 0) & (j == 0))
    def _init():
        acc_ref[0] = 0.0
        acc_ref[1] = 0.0
        acc_ref[2] = 0.0

    blk = lp_ref[...]
    tgt = tgt_ref[...]
    valid = (tgt != 1).astype(jnp.float32)

    rowsum = jnp.sum(blk, axis=1, keepdims=True)
    acc_ref[0] += jnp.sum(rowsum * valid)

    tgtc = jnp.maximum(tgt, 0)
    hit = col_ref[...] == tgtc
    acc_ref[1] += jnp.sum(jnp.where(hit, blk, 0.0) * valid)

    @pl.when(j == 0)
    def _count():
        acc_ref[2] += jnp.sum(valid)

    @pl.when((i == pl.num_programs(0) - 1) & (j == pl.num_programs(1) - 1))
    def _finalize():
        c1 = _LS / _V
        c2 = 1.0 - _LS - c1
        denom = jnp.maximum(acc_ref[2], 1.0)
        out_ref[0, 0] = -(c1 * acc_ref[0] + c2 * acc_ref[1]) / denom


def kernel(log_probs, targets, triplets):
    tgt2d = jnp.asarray(targets, jnp.int32).reshape(_N, 1)
    cols = jnp.arange(_V, dtype=jnp.int32).reshape(1, _V)
    out = pl.pallas_call(
        _body,
        grid=(_N // _RB, _V // _CB),
        in_specs=[
            pl.BlockSpec((_RB, 1), lambda i, j: (i, 0)),
            pl.BlockSpec((1, _CB), lambda i, j: (0, j)),
            pl.BlockSpec((_RB, _CB), lambda i, j: (i, j)),
        ],
        out_specs=pl.BlockSpec(memory_space=pltpu.SMEM),
        out_shape=jax.ShapeDtypeStruct((1, 1), jnp.float32),
        scratch_shapes=[pltpu.SMEM((3,), jnp.float32)],
    )(tgt2d, cols, log_probs)
    return out[0, 0]


# speedup vs baseline: 3.3053x; 1.0586x over previous
import jax
import jax.numpy as jnp
from jax.experimental import pallas as pl
from jax.experimental.pallas import tpu as pltpu

_N = 4096
_V = 32000
_LS = 0.1
_RB = 1024
_CB = 6400


def _body(tgt_ref, col_ref, lp_ref, out_ref, acc_ref):
    i = pl.program_id(0)
    j = pl.program_id(1)

    @pl.when((i == 0) & (j == 0))
    def _init():
        acc_ref[0] = 0.0
        acc_ref[1] = 0.0
        acc_ref[2] = 0.0

    blk = lp_ref[...]
    tgt = tgt_ref[...]
    valid = (tgt != 1).astype(jnp.float32)

    rowsum = jnp.sum(blk, axis=1, keepdims=True)
    acc_ref[0] += jnp.sum(rowsum * valid)

    tgtc = jnp.maximum(tgt, 0)
    hit = col_ref[...] == tgtc
    s2row = jnp.sum(jnp.where(hit, blk, 0.0), axis=1, keepdims=True)
    acc_ref[1] += jnp.sum(s2row * valid)

    @pl.when(j == 0)
    def _count():
        acc_ref[2] += jnp.sum(valid)

    @pl.when((i == pl.num_programs(0) - 1) & (j == pl.num_programs(1) - 1))
    def _finalize():
        c1 = _LS / _V
        c2 = 1.0 - _LS - c1
        denom = jnp.maximum(acc_ref[2], 1.0)
        out_ref[0, 0] = -(c1 * acc_ref[0] + c2 * acc_ref[1]) / denom


def kernel(log_probs, targets, triplets):
    tgt2d = jnp.asarray(targets, jnp.int32).reshape(_N, 1)
    cols = jnp.arange(_V, dtype=jnp.int32).reshape(1, _V)
    out = pl.pallas_call(
        _body,
        grid=(_N // _RB, _V // _CB),
        in_specs=[
            pl.BlockSpec((_RB, 1), lambda i, j: (i, 0)),
            pl.BlockSpec((1, _CB), lambda i, j: (0, j)),
            pl.BlockSpec((_RB, _CB), lambda i, j: (i, j)),
        ],
        out_specs=pl.BlockSpec(memory_space=pltpu.SMEM),
        out_shape=jax.ShapeDtypeStruct((1, 1), jnp.float32),
        scratch_shapes=[pltpu.SMEM((3,), jnp.float32)],
    )(tgt2d, cols, log_probs)
    return out[0, 0]


# speedup vs baseline: 3.3478x; 1.0129x over previous
import jax
import jax.numpy as jnp
from jax.experimental import pallas as pl
from jax.experimental.pallas import tpu as pltpu

_N = 4096
_V = 32000
_LS = 0.1
_RB = 512
_CB = 6400


def _body(tgt_ref, col_ref, lp_ref, out_ref, acc_ref):
    i = pl.program_id(0)
    j = pl.program_id(1)

    @pl.when((i == 0) & (j == 0))
    def _init():
        acc_ref[0] = 0.0
        acc_ref[1] = 0.0
        acc_ref[2] = 0.0

    blk = lp_ref[...]
    tgt = tgt_ref[...]
    valid = (tgt != 1).astype(jnp.float32)

    rowsum = jnp.sum(blk, axis=1, keepdims=True)
    acc_ref[0] += jnp.sum(rowsum * valid)

    tgtc = jnp.maximum(tgt, 0)
    hit = col_ref[...] == tgtc
    s2row = jnp.sum(jnp.where(hit, blk, 0.0), axis=1, keepdims=True)
    acc_ref[1] += jnp.sum(s2row * valid)

    @pl.when(j == 0)
    def _count():
        acc_ref[2] += jnp.sum(valid)

    @pl.when((i == pl.num_programs(0) - 1) & (j == pl.num_programs(1) - 1))
    def _finalize():
        c1 = _LS / _V
        c2 = 1.0 - _LS - c1
        denom = jnp.maximum(acc_ref[2], 1.0)
        out_ref[0, 0] = -(c1 * acc_ref[0] + c2 * acc_ref[1]) / denom


def kernel(log_probs, targets, triplets):
    tgt2d = jnp.asarray(targets, jnp.int32).reshape(_N, 1)
    cols = jnp.arange(_V, dtype=jnp.int32).reshape(1, _V)
    out = pl.pallas_call(
        _body,
        grid=(_N // _RB, _V // _CB),
        in_specs=[
            pl.BlockSpec((_RB, 1), lambda i, j: (i, 0)),
            pl.BlockSpec((1, _CB), lambda i, j: (0, j)),
            pl.BlockSpec((_RB, _CB), lambda i, j: (i, j)),
        ],
        out_specs=pl.BlockSpec(memory_space=pltpu.SMEM),
        out_shape=jax.ShapeDtypeStruct((1, 1), jnp.float32),
        scratch_shapes=[pltpu.SMEM((3,), jnp.float32)],
    )(tgt2d, cols, log_probs)
    return out[0, 0]
